# baseline (speedup 1.0000x reference)
import functools

import jax
import jax.numpy as jnp
from jax import lax
from jax.experimental import pallas as pl
from jax.experimental.pallas import tpu as pltpu
from jax.experimental.pallas import tpu_sc as plsc

_NW = 32
_VOCAB = 1000001
_VP = 1000064
_NCOLS = _VP // 128
_COLS_PW = 246
_PACK_ROWS = _VP // 2
_D = 64
_MAX_POS = 1024

_CPARAMS = pltpu.CompilerParams(use_tc_tiling_on_sc=True, needs_layout_passes=False)


def _pos_freqs(nt: int) -> jnp.ndarray:
    dim = _D
    inv = 1.0 / (10000.0 ** (jnp.arange(0, dim, 2)[: dim // 2].astype(jnp.float32) / dim))
    pos = jnp.minimum(jnp.arange(nt, dtype=jnp.float32), float(_MAX_POS - 1))
    f = pos[:, None] * inv[None, :]
    return jnp.concatenate([jnp.cos(f), jnp.sin(f)], axis=-1)


_TC_COLS = 512
_TC_GRID = 1954
_PACK_PAD = _TC_GRID * _TC_COLS // 2


def _tc_pack_block(wt_blk, packed_blk):
    y = jnp.transpose(wt_blk[...])
    packed_blk[...] = jnp.concatenate([y[:256], y[256:]], axis=1)


def _gather_body(p_hbm, u_hbm, q_hbm, fq_hbm, out_hbm, u_v, q_v, fq_v,
                 g0, g1, g2, o0, o1, o2,
                 sg0, sg1, sg2, so0, so1, so2):
    wid = lax.axis_index("s") * 2 + lax.axis_index("c")
    t0 = (wid * 50) // 8
    pltpu.sync_copy(u_hbm.at[wid], u_v)
    pltpu.sync_copy(q_hbm.at[wid], q_v)
    pltpu.sync_copy(fq_hbm.at[pl.ds(t0, 8)], fq_v)
    iota = lax.iota(jnp.int32, 16)
    rows_g = [iota + 16 * g for g in range(8)]

    gbuf = (g0, g1, g2)
    obuf = (o0, o1, o2)
    gsem = (sg0, sg1, sg2)
    osem = (so0, so1, so2)

    def start_gather(k, b):
        pltpu.async_copy(p_hbm.at[u_v.at[k]], gbuf[b], gsem[b])

    def wait_gather(b):
        pltpu.make_async_copy(p_hbm.at[u_v.at[0]], gbuf[b], gsem[b]).wait()

    def wait_out(b):
        pltpu.make_async_copy(p_hbm.at[pl.ds(0, 64)], obuf[b], osem[b]).wait()

    start_gather(0, 0)
    start_gather(1, 1)

    def blk(k, b, first):
        blk_id = wid * 50 + k
        t = blk_id // 8
        bb = lax.rem(blk_id, 8)

        @pl.when(k + 2 < 50)
        def _():
            start_gather(k + 2, (b + 2) % 3)

        wait_gather(b)
        if not first:
            wait_out(b)
        tl = t - t0
        par64 = [q_v[k, pl.ds(g * 16, 16)] for g in range(8)]

        def d_body(d, carry):
            dvec = jnp.full((16,), d, jnp.int32)
            fqs = fq_v[tl, d // 8, pl.ds(lax.rem(d, 8) * 16, 16)]
            vecs = [plsc.load_gather(gbuf[b], [rows_g[g], par64[g] + dvec])
                    for g in range(8)]
            for g in range(8):
                obuf[b][d, pl.ds(g * 16, 16)] = vecs[g] + fqs
            return carry

        lax.fori_loop(0, 64, d_body, 0, unroll=4)
        pltpu.async_copy(obuf[b], out_hbm.at[t, :, pl.ds(bb * 128, 128)], osem[b])

    blk(0, 0, first=True)
    blk(1, 1, first=True)
    blk(2, 2, first=True)

    def tri(i, carry):
        for j in range(3):
            blk(3 * i + j, j, first=False)
        return carry

    lax.fori_loop(1, 16, tri, 0)
    blk(48, 0, first=False)
    blk(49, 1, first=False)
    wait_out(0)
    wait_out(1)
    wait_out(2)


def kernel(text, text_embed_weight):
    b, nt = text.shape
    vocab, d = text_embed_weight.shape
    assert (b, nt, vocab, d) == (1024, 200, _VOCAB, _D)

    mesh = plsc.VectorSubcoreMesh(core_axis_name="c", subcore_axis_name="s")

    wt = text_embed_weight.T
    v3 = (text.astype(jnp.int32) + 1).T.reshape(_NW, 50, 128)
    u3 = (v3 >> 9) * 256 + (v3 & 255)
    q3 = ((v3 >> 8) & 1) * 64
    fqb = jnp.pad(jnp.repeat(_pos_freqs(nt), 16, axis=1).reshape(nt, 8, 128),
                  ((0, 8), (0, 0), (0, 0)))

    packed = pl.pallas_call(
        _tc_pack_block,
        grid=(_TC_GRID,),
        in_specs=[pl.BlockSpec((_D, _TC_COLS), lambda i: (0, i))],
        out_specs=pl.BlockSpec((_TC_COLS // 2, 128), lambda i: (i, 0)),
        out_shape=jax.ShapeDtypeStruct((_PACK_PAD, 128), jnp.float32),
    )(wt)

    gather = functools.partial(
        pl.kernel,
        mesh=mesh,
        compiler_params=_CPARAMS,
        out_type=jax.ShapeDtypeStruct((nt, d, b), jnp.float32),
        scratch_types=[
            pltpu.VMEM((50, 128), jnp.int32),
            pltpu.VMEM((50, 128), jnp.int32),
            pltpu.VMEM((8, 8, 128), jnp.float32),
            pltpu.VMEM((128, 128), jnp.float32),
            pltpu.VMEM((128, 128), jnp.float32),
            pltpu.VMEM((128, 128), jnp.float32),
            pltpu.VMEM((64, 128), jnp.float32),
            pltpu.VMEM((64, 128), jnp.float32),
            pltpu.VMEM((64, 128), jnp.float32),
            pltpu.SemaphoreType.DMA,
            pltpu.SemaphoreType.DMA,
            pltpu.SemaphoreType.DMA,
            pltpu.SemaphoreType.DMA,
            pltpu.SemaphoreType.DMA,
            pltpu.SemaphoreType.DMA,
        ],
    )(_gather_body)
    out = gather(packed, u3, q3, fqb)
    return jnp.transpose(out, (2, 0, 1))


# speedup vs baseline: 1.2474x; 1.2474x over previous
import functools

import jax
import jax.numpy as jnp
from jax import lax
from jax.experimental import pallas as pl
from jax.experimental.pallas import tpu as pltpu
from jax.experimental.pallas import tpu_sc as plsc

_NW = 32
_VOCAB = 1000001
_VP = 1000064
_NCOLS = _VP // 128
_COLS_PW = 246
_PACK_ROWS = _VP // 2
_D = 64
_MAX_POS = 1024

_CPARAMS = pltpu.CompilerParams(use_tc_tiling_on_sc=True, needs_layout_passes=False)


def _pos_freqs(nt: int) -> jnp.ndarray:
    dim = _D
    inv = 1.0 / (10000.0 ** (jnp.arange(0, dim, 2)[: dim // 2].astype(jnp.float32) / dim))
    pos = jnp.minimum(jnp.arange(nt, dtype=jnp.float32), float(_MAX_POS - 1))
    f = pos[:, None] * inv[None, :]
    return jnp.concatenate([jnp.cos(f), jnp.sin(f)], axis=-1)


_TC_COLS = 512
_TC_GRID = 1954
_PACK_PAD = _TC_GRID * _TC_COLS // 2


def _tc_pack_block(wt_blk, packed_blk):
    y = jnp.transpose(wt_blk[...])
    packed_blk[...] = jnp.concatenate([y[:256], y[256:]], axis=1)


def _gather_body(p_hbm, u_hbm, q_hbm, fq_hbm, out_hbm, u_v, q_v, fq_v,
                 g0, g1, g2, o0, o1, o2,
                 sg0, sg1, sg2, so0, so1, so2):
    wid = lax.axis_index("s") * 2 + lax.axis_index("c")
    t0 = (wid * 50) // 8
    pltpu.sync_copy(u_hbm.at[wid], u_v)
    pltpu.sync_copy(q_hbm.at[wid], q_v)
    pltpu.sync_copy(fq_hbm.at[pl.ds(t0, 8)], fq_v)
    iota = lax.iota(jnp.int32, 16)
    rows_g = [iota + 16 * g for g in range(8)]

    gbuf = (g0, g1, g2)
    obuf = (o0, o1, o2)
    gsem = (sg0, sg1, sg2)
    osem = (so0, so1, so2)

    def start_gather(k, b):
        pltpu.async_copy(p_hbm.at[u_v.at[k]], gbuf[b], gsem[b])

    def wait_gather(b):
        pltpu.make_async_copy(p_hbm.at[u_v.at[0]], gbuf[b], gsem[b]).wait()

    def wait_out(b):
        pltpu.make_async_copy(p_hbm.at[pl.ds(0, 64)], obuf[b], osem[b]).wait()

    start_gather(0, 0)
    start_gather(1, 1)

    def blk(k, b, first):
        blk_id = wid * 50 + k
        t = blk_id // 8
        bb = lax.rem(blk_id, 8)

        @pl.when(k + 2 < 50)
        def _():
            start_gather(k + 2, (b + 2) % 3)

        wait_gather(b)
        if not first:
            wait_out(b)
        tl = t - t0
        par64 = [q_v[k, pl.ds(g * 16, 16)] for g in range(8)]

        def d_body(d, carry):
            dvec = jnp.full((16,), d, jnp.int32)
            fqs = fq_v[tl, d // 8, pl.ds(lax.rem(d, 8) * 16, 16)]
            vecs = [plsc.load_gather(gbuf[b], [rows_g[g], par64[g] + dvec])
                    for g in range(8)]
            for g in range(8):
                obuf[b][d, pl.ds(g * 16, 16)] = vecs[g] + fqs
            return carry

        lax.fori_loop(0, 64, d_body, 0, unroll=4)
        pltpu.async_copy(obuf[b], out_hbm.at[t, :, pl.ds(bb * 128, 128)], osem[b])

    blk(0, 0, first=True)
    blk(1, 1, first=True)
    blk(2, 2, first=True)

    def tri(i, carry):
        for j in range(3):
            blk(3 * i + j, j, first=False)
        return carry

    lax.fori_loop(1, 16, tri, 0)
    blk(48, 0, first=False)
    blk(49, 1, first=False)
    wait_out(0)
    wait_out(1)
    wait_out(2)


def kernel(text, text_embed_weight):
    b, nt = text.shape
    vocab, d = text_embed_weight.shape
    assert (b, nt, vocab, d) == (1024, 200, _VOCAB, _D)

    mesh = plsc.VectorSubcoreMesh(core_axis_name="c", subcore_axis_name="s")

    v3 = (text.astype(jnp.int32) + 1).T.reshape(_NW, 50, 128)
    u3 = v3 >> 1
    q3 = (v3 & 1) * 64
    fqb = jnp.pad(jnp.repeat(_pos_freqs(nt), 16, axis=1).reshape(nt, 8, 128),
                  ((0, 8), (0, 0), (0, 0)))

    packed = jnp.pad(text_embed_weight, ((0, 1), (0, 0))).reshape(
        (_VOCAB + 1) // 2, 2 * _D)

    gather = functools.partial(
        pl.kernel,
        mesh=mesh,
        compiler_params=_CPARAMS,
        out_type=jax.ShapeDtypeStruct((nt, d, b), jnp.float32),
        scratch_types=[
            pltpu.VMEM((50, 128), jnp.int32),
            pltpu.VMEM((50, 128), jnp.int32),
            pltpu.VMEM((8, 8, 128), jnp.float32),
            pltpu.VMEM((128, 128), jnp.float32),
            pltpu.VMEM((128, 128), jnp.float32),
            pltpu.VMEM((128, 128), jnp.float32),
            pltpu.VMEM((64, 128), jnp.float32),
            pltpu.VMEM((64, 128), jnp.float32),
            pltpu.VMEM((64, 128), jnp.float32),
            pltpu.SemaphoreType.DMA,
            pltpu.SemaphoreType.DMA,
            pltpu.SemaphoreType.DMA,
            pltpu.SemaphoreType.DMA,
            pltpu.SemaphoreType.DMA,
            pltpu.SemaphoreType.DMA,
        ],
    )(_gather_body)
    out = gather(packed, u3, q3, fqb)
    return jnp.transpose(out, (2, 0, 1))


# speedup vs baseline: 1.6931x; 1.3573x over previous
import functools

import jax
import jax.numpy as jnp
from jax import lax
from jax.experimental import pallas as pl
from jax.experimental.pallas import tpu as pltpu
from jax.experimental.pallas import tpu_sc as plsc

_OUT_D = 64
_MAX_POS = 1024
_CHUNK = 128


def _pos_freqs(nt: int) -> jnp.ndarray:
    dim = _OUT_D
    inv = 1.0 / (10000.0 ** (jnp.arange(0, dim, 2)[: dim // 2].astype(jnp.float32) / dim))
    pos = jnp.minimum(jnp.arange(nt, dtype=jnp.float32), float(_MAX_POS - 1))
    f = pos[:, None] * inv[None, :]
    return jnp.concatenate([jnp.cos(f), jnp.sin(f)], axis=-1)


def kernel(text, text_embed_weight):
    b, nt = text.shape
    d = text_embed_weight.shape[1]
    total = b * nt

    info = plsc.get_sparse_core_info()
    nc, ns = info.num_cores, info.num_subcores
    nw = nc * ns
    per_w = total // nw
    nchunk = per_w // _CHUNK
    assert d == _OUT_D
    assert per_w * nw == total
    assert nchunk * _CHUNK == per_w
    assert per_w % nt == 0
    assert nchunk % 3 == 0 or nchunk % 3 == 2

    fq2 = jnp.concatenate([_pos_freqs(nt)] * 2, axis=0)

    idx = text.reshape(nw, nchunk, _CHUNK).astype(jnp.int32)

    mesh = plsc.VectorSubcoreMesh(core_axis_name="c", subcore_axis_name="s")

    @functools.partial(
        pl.kernel,
        mesh=mesh,
        compiler_params=pltpu.CompilerParams(use_tc_tiling_on_sc=False),
        out_type=jax.ShapeDtypeStruct((total, d), jnp.float32),
        scratch_types=[
            pltpu.VMEM((nchunk, _CHUNK), jnp.int32),
            pltpu.VMEM((2 * nt, d), jnp.float32),
            pltpu.VMEM((_CHUNK, d), jnp.float32),
            pltpu.VMEM((_CHUNK, d), jnp.float32),
            pltpu.VMEM((_CHUNK, d), jnp.float32),
            pltpu.SemaphoreType.DMA,
            pltpu.SemaphoreType.DMA,
            pltpu.SemaphoreType.DMA,
        ],
    )
    def emb_kernel(w_hbm, idx_hbm, fq_hbm, out_hbm, idx_v, fq_v,
                   buf0, buf1, buf2, sem0, sem1, sem2):
        wid = lax.axis_index("s") * nc + lax.axis_index("c")
        base = wid * per_w
        pltpu.sync_copy(idx_hbm.at[wid], idx_v)
        pltpu.sync_copy(fq_hbm, fq_v)

        bufs = (buf0, buf1, buf2)
        sems = (sem0, sem1, sem2)

        def bump(row):
            for c in range(_CHUNK // 16):
                sl = pl.ds(c * 16, 16)
                idx_v[row, sl] = idx_v[row, sl] + 1

        def start_gather(k, bf):
            pltpu.async_copy(w_hbm.at[idx_v.at[k]], bufs[bf], sems[bf])

        def wait_gather(bf):
            pltpu.make_async_copy(w_hbm.at[idx_v.at[0]], bufs[bf],
                                  sems[bf]).wait()

        bump(0)
        start_gather(0, 0)
        bump(1)
        start_gather(1, 1)

        def chunk_body(k, bf):
            @pl.when(k + 2 < nchunk)
            def _():
                bump(k + 2)
                start_gather(k + 2, (bf + 2) % 3)

            wait_gather(bf)
            buf = bufs[bf]
            t0 = lax.rem(k * _CHUNK, nt)

            def row_body(r, carry):
                t = t0 + r
                for c in range(d // 16):
                    sl = pl.ds(c * 16, 16)
                    buf[r, sl] = buf[r, sl] + fq_v[t, sl]
                return carry

            lax.fori_loop(0, _CHUNK, row_body, 0, unroll=4)
            pltpu.sync_copy(buf, out_hbm.at[pl.ds(base + k * _CHUNK, _CHUNK)])

        def outer(i, carry):
            chunk_body(3 * i, 0)
            chunk_body(3 * i + 1, 1)
            chunk_body(3 * i + 2, 2)
            return carry

        lax.fori_loop(0, nchunk // 3, outer, 0)
        if nchunk % 3 == 2:
            chunk_body(nchunk - 2, 0)
            chunk_body(nchunk - 1, 1)

    out = emb_kernel(text_embed_weight, idx, fq2)
    return out.reshape(b, nt, d)
